# baseline (speedup 1.0000x reference)
import functools

import jax
import jax.numpy as jnp
from jax import lax
from jax.experimental import pallas as pl
from jax.experimental.pallas import tpu as pltpu
from jax.experimental.pallas import tpu_sc as plsc

FEATURE_NUM = 1040000
LATENT_DIM = 16
FIELD_NUM = 26
BATCH = 16384

N = BATCH * FIELD_NUM
NC = 2
NS = 16
NW = NC * NS
PER_W = N // NW
GATHER_ROWS = 128
CHUNK = 1664
KJ = CHUNK // GATHER_ROWS
NCHUNK = PER_W // CHUNK

_mesh = plsc.VectorSubcoreMesh(core_axis_name="c", subcore_axis_name="s")


@functools.partial(
    pl.kernel,
    out_type=jax.ShapeDtypeStruct((N, LATENT_DIM), jnp.float32),
    mesh=_mesh,
    compiler_params=pltpu.CompilerParams(
        needs_layout_passes=False, use_tc_tiling_on_sc=False
    ),
    scratch_types=[
        pltpu.VMEM((CHUNK,), jnp.int32),
        pltpu.VMEM((CHUNK, LATENT_DIM), jnp.float32),
        pltpu.VMEM((CHUNK,), jnp.float32),
        pltpu.SemaphoreType.DMA,
    ],
)
def _sc_embed(xidx_hbm, thr_hbm, table_hbm, out_hbm, idx_v, rows_v, thr_v, sem):
    wid = lax.axis_index("s") * NC + lax.axis_index("c")
    base = wid * PER_W

    pltpu.sync_copy(thr_hbm, thr_v)

    def chunk_body(c, carry):
        start = base + c * CHUNK
        pltpu.sync_copy(xidx_hbm.at[pl.ds(start, CHUNK)], idx_v)

        descs = []
        for j in range(KJ):
            descs.append(
                pltpu.async_copy(
                    table_hbm.at[idx_v.at[pl.ds(j * GATHER_ROWS, GATHER_ROWS)]],
                    rows_v.at[pl.ds(j * GATHER_ROWS, GATHER_ROWS)],
                    sem,
                )
            )
        for d in descs:
            d.wait()

        def blk_body(blk, rcarry):
            r0 = blk * 16
            ridx = r0 + lax.iota(jnp.int32, 16)
            sums = jnp.zeros((16,), jnp.float32)
            cols = []
            for d in range(LATENT_DIM):
                cidx = jnp.full((16,), d, jnp.int32)
                col = plsc.load_gather(rows_v, [ridx, cidx])
                cols.append(col)
                sums = sums + jnp.abs(col)
            t = thr_v[pl.ds(r0, 16)]
            m = ((sums - t) > 0).astype(jnp.float32)
            for d in range(LATENT_DIM):
                cidx = jnp.full((16,), d, jnp.int32)
                plsc.store_scatter(rows_v, [ridx, cidx], cols[d] * m)
            return rcarry

        lax.fori_loop(0, CHUNK // 16, blk_body, 0)

        pltpu.sync_copy(rows_v, out_hbm.at[pl.ds(start, CHUNK)])
        return carry

    lax.fori_loop(0, NCHUNK, chunk_body, 0)


@jax.jit
def kernel(x, phase, embedding, threshold):
    xflat = x.reshape(N)
    thr_rep = jnp.tile(threshold.reshape(FIELD_NUM), CHUNK // FIELD_NUM)
    out = _sc_embed(xflat, thr_rep, embedding)
    return out.reshape(BATCH, FIELD_NUM, LATENT_DIM)


# speedup vs baseline: 1.7804x; 1.7804x over previous
import functools

import jax
import jax.numpy as jnp
from jax import lax
from jax.experimental import pallas as pl
from jax.experimental.pallas import tpu as pltpu
from jax.experimental.pallas import tpu_sc as plsc

FEATURE_NUM = 1040000
LATENT_DIM = 16
FIELD_NUM = 26
BATCH = 16384

NC = 2
NS = 16
NW = NC * NS
BW = BATCH // NW
GATHER_ROWS = 128
KJ = BW // GATHER_ROWS
NBLK = BW // 16
NT = BATCH // 128
TW = NT // NW

_mesh = plsc.VectorSubcoreMesh(core_axis_name="c", subcore_axis_name="s")


@functools.partial(
    pl.kernel,
    out_type=jax.ShapeDtypeStruct(
        (FIELD_NUM, LATENT_DIM // 8, NT, 8, 128), jnp.float32
    ),
    mesh=_mesh,
    compiler_params=pltpu.CompilerParams(
        needs_layout_passes=False, use_tc_tiling_on_sc=False
    ),
    scratch_types=[
        pltpu.VMEM((BW,), jnp.int32),
        pltpu.VMEM((BW, LATENT_DIM), jnp.float32),
        pltpu.VMEM((2, TW, 8, 128), jnp.float32),
        pltpu.VMEM((FIELD_NUM, 16), jnp.float32),
        pltpu.SemaphoreType.DMA,
    ],
)
def _sc_embed(xt_hbm, thr_hbm, table_hbm, out_hbm, idx_v, rows_v, trans_v,
              thr_v, sem):
    wid = lax.axis_index("s") * NC + lax.axis_index("c")
    b0 = wid * BW
    t0 = wid * TW

    pltpu.sync_copy(thr_hbm, thr_v)

    def field_body(f, carry):
        pltpu.sync_copy(xt_hbm.at[f, pl.ds(b0, BW)], idx_v)

        descs = []
        for j in range(KJ):
            descs.append(
                pltpu.async_copy(
                    table_hbm.at[idx_v.at[pl.ds(j * GATHER_ROWS, GATHER_ROWS)]],
                    rows_v.at[pl.ds(j * GATHER_ROWS, GATHER_ROWS)],
                    sem,
                )
            )
        for d in descs:
            d.wait()

        t_vec = thr_v[f, :]

        def blk_body(blk, rcarry):
            ridx = blk * 16 + lax.iota(jnp.int32, 16)
            sums = jnp.zeros((16,), jnp.float32)
            cols = []
            for d in range(LATENT_DIM):
                cidx = jnp.full((16,), d, jnp.int32)
                col = plsc.load_gather(rows_v, [ridx, cidx])
                cols.append(col)
                sums = sums + jnp.abs(col)
            m = ((sums - t_vec) > 0).astype(jnp.float32)
            tl = blk // 8
            rr0 = (blk % 8) * 16
            for d in range(LATENT_DIM):
                trans_v[d // 8, tl, d % 8, pl.ds(rr0, 16)] = cols[d] * m
            return rcarry

        lax.fori_loop(0, NBLK, blk_body, 0)

        for g in range(2):
            pltpu.sync_copy(trans_v.at[g], out_hbm.at[f, g, pl.ds(t0, TW)])
        return carry

    lax.fori_loop(0, FIELD_NUM, field_body, 0)


@jax.jit
def kernel(x, phase, embedding, threshold):
    xt = x.T
    thr = jnp.broadcast_to(threshold, (FIELD_NUM, 16))
    out5 = _sc_embed(xt, thr, embedding)
    return out5.transpose(2, 4, 0, 1, 3).reshape(BATCH, FIELD_NUM, LATENT_DIM)
